# baseline (speedup 1.0000x reference)
import jax
import jax.numpy as jnp
from jax.experimental import pallas as pl
from jax.experimental.pallas import tpu as pltpu

C_CHUNK = 2048
G_EXTRA = 2048
S_TOTAL = 2 * C_CHUNK + G_EXTRA
TAIL = S_TOTAL - C_CHUNK
PIECE = 512
N_TP = TAIL // PIECE
N_CP = C_CHUNK // PIECE


def _body(k_ref, v_ref, fk_ref, K_ref, V_ref, FK_ref,
          zkv, zfk, sk, sv, sf, sem_z, sem_i, sem_o):
    B = k_ref.shape[0]
    zkv[...] = jnp.zeros(zkv.shape, zkv.dtype)
    zfk[...] = jnp.zeros(zfk.shape, zfk.dtype)

    zeros = []
    for b in range(B):
        for t in range(N_TP):
            s0 = C_CHUNK + t * PIECE
            zeros.append(pltpu.make_async_copy(zkv, K_ref.at[b, pl.ds(s0, PIECE)], sem_z))
            zeros.append(pltpu.make_async_copy(zkv, V_ref.at[b, pl.ds(s0, PIECE)], sem_z))
            zeros.append(pltpu.make_async_copy(zfk, FK_ref.at[b, pl.ds(s0, PIECE)], sem_z))
    for c in zeros:
        c.start()

    pieces = [(b, t * PIECE) for b in range(B) for t in range(N_CP)]
    arrs = []
    for src, dst, stage, si, so in (
            (k_ref, K_ref, sk, sem_i.at[0], sem_o.at[0]),
            (v_ref, V_ref, sv, sem_i.at[1], sem_o.at[1]),
            (fk_ref, FK_ref, sf, sem_i.at[2], sem_o.at[2])):
        ins, outs = [], []
        for p, (b, s0) in enumerate(pieces):
            ins.append(pltpu.make_async_copy(
                src.at[b, pl.ds(s0, PIECE)], stage.at[p % 2], si))
            outs.append(pltpu.make_async_copy(
                stage.at[p % 2], dst.at[b, pl.ds(s0, PIECE)], so))
        arrs.append((ins, outs))

    for ins, outs in arrs:
        ins[0].start()
        ins[1].start()
    n = len(pieces)
    for p in range(n):
        for ins, outs in arrs:
            ins[p].wait()
            outs[p].start()
        if p + 2 < n:
            for ins, outs in arrs:
                outs[p].wait()
                ins[p + 2].start()
    for ins, outs in arrs:
        outs[n - 2].wait()
        outs[n - 1].wait()
    for c in zeros:
        c.wait()


def kernel(k_c, v_c, fk_c):
    B, C, H, D = k_c.shape
    F = fk_c.shape[-1]

    K, V, FK = pl.pallas_call(
        _body,
        in_specs=[pl.BlockSpec(memory_space=pl.ANY)] * 3,
        out_specs=[pl.BlockSpec(memory_space=pl.ANY)] * 3,
        out_shape=[
            jax.ShapeDtypeStruct((B, S_TOTAL, H, D), k_c.dtype),
            jax.ShapeDtypeStruct((B, S_TOTAL, H, D), v_c.dtype),
            jax.ShapeDtypeStruct((B, S_TOTAL, H, F), fk_c.dtype),
        ],
        scratch_shapes=[
            pltpu.VMEM((PIECE, H, D), k_c.dtype),
            pltpu.VMEM((PIECE, H, F), fk_c.dtype),
            pltpu.VMEM((2, PIECE, H, D), k_c.dtype),
            pltpu.VMEM((2, PIECE, H, D), v_c.dtype),
            pltpu.VMEM((2, PIECE, H, F), fk_c.dtype),
            pltpu.SemaphoreType.DMA,
            pltpu.SemaphoreType.DMA((3,)),
            pltpu.SemaphoreType.DMA((3,)),
        ],
    )(k_c, v_c, fk_c)

    Hs = jnp.zeros((B, H, F, D), dtype=k_c.dtype)
    S = jnp.zeros((B, H, F), dtype=k_c.dtype)
    return (K, V, FK, Hs, S)
